# baseline (speedup 1.0000x reference)
import functools

import jax
import jax.numpy as jnp
from jax import lax
from jax.experimental import pallas as pl
from jax.experimental.pallas import tpu as pltpu
from jax.experimental.pallas import tpu_sc as plsc

_VOCAB = 100000
_PRE_DIM = 64
_EMB_DIM = 64
_HIDDEN = 256
_BATCH = 16384

_NC = 2
_NS = 16
_NW = _NC * _NS
_CPW = _PRE_DIM // _NW
_OCHUNK = 4096


def _sc_gather_t(idx, table_t):
    mesh = plsc.VectorSubcoreMesh(core_axis_name="c", subcore_axis_name="s")

    @functools.partial(
        pl.kernel,
        out_type=jax.ShapeDtypeStruct((_PRE_DIM, _BATCH), jnp.float32),
        mesh=mesh,
        compiler_params=pltpu.CompilerParams(needs_layout_passes=False),
        scratch_types=[
            pltpu.VMEM((_BATCH,), jnp.int32),
            pltpu.VMEM((_VOCAB + 1,), jnp.float32),
            pltpu.VMEM((2, _OCHUNK), jnp.float32),
            pltpu.VMEM_SHARED((_BATCH,), jnp.int32),
            pltpu.SemaphoreType.DMA,
            pltpu.SemaphoreType.DMA,
        ],
    )
    def k(idx_hbm, table_hbm, out_hbm, idx_v, col_v, obuf_v, idx_sp, sem, osem):
        sid = lax.axis_index("s")
        wid = sid * _NC + lax.axis_index("c")
        col_cp = pltpu.async_copy(table_hbm.at[wid * _CPW], col_v, sem)
        @pl.when(sid == 0)
        def _():
            pltpu.sync_copy(idx_hbm, idx_sp)

        plsc.subcore_barrier()
        pltpu.sync_copy(idx_sp, idx_v)

        def drain_one():
            pltpu.make_async_copy(
                out_hbm.at[0].at[pl.ds(0, _OCHUNK)],
                obuf_v.at[0],
                osem,
            ).wait()

        nhalf = _BATCH // _OCHUNK
        for kk in range(_CPW * nhalf):
            c, half = divmod(kk, nhalf)
            col = wid * _CPW + c
            if half == 0:
                col_cp.wait()

            def gather_body(i, carry, half=half, kk=kk):
                for u in range(4):
                    si = pl.ds(
                        pl.multiple_of(half * _OCHUNK + i * 64 + u * 16, 16),
                        16)
                    so = pl.ds(pl.multiple_of(i * 64 + u * 16, 16), 16)
                    v = idx_v[si]
                    vi = jnp.where((v >= 1) & (v <= _VOCAB), v, 0)
                    obuf_v[kk % 2, so] = plsc.load_gather(col_v, [vi])
                return carry

            if kk >= 2:
                drain_one()
            lax.fori_loop(0, _OCHUNK // 64, gather_body, 0)
            pltpu.async_copy(
                obuf_v.at[kk % 2],
                out_hbm.at[col].at[pl.ds(half * _OCHUNK, _OCHUNK)],
                osem,
            )
            if half == nhalf - 1 and c + 1 < _CPW:
                col_cp = pltpu.async_copy(table_hbm.at[col + 1], col_v, sem)
        drain_one()
        drain_one()

    return k(idx, table_t)


def _mlp_body(xt_ref, w1_ref, b1_ref, w2_ref, b2_ref, ot_ref):
    xt = xt_ref[...]
    h = jax.lax.dot_general(
        xt, w1_ref[...], (((0,), (0,)), ((), ())),
        preferred_element_type=jnp.float32)
    h = jnp.maximum(h + b1_ref[...], 0.0)
    ot = jax.lax.dot_general(
        w2_ref[...], h, (((0,), (1,)), ((), ())),
        preferred_element_type=jnp.float32)
    ot_ref[...] = ot + b2_ref[...]


def _tc_mlp_t(embt, W1, b1, W2, b2t):
    tm = 2048
    grid = (_BATCH // tm,)
    return pl.pallas_call(
        _mlp_body,
        grid=grid,
        in_specs=[
            pl.BlockSpec((_PRE_DIM, tm), lambda i: (0, i)),
            pl.BlockSpec((_PRE_DIM, _HIDDEN), lambda i: (0, 0)),
            pl.BlockSpec((1, _HIDDEN), lambda i: (0, 0)),
            pl.BlockSpec((_HIDDEN, _EMB_DIM), lambda i: (0, 0)),
            pl.BlockSpec((_EMB_DIM, 1), lambda i: (0, 0)),
        ],
        out_specs=pl.BlockSpec((_EMB_DIM, tm), lambda i: (0, i)),
        out_shape=jax.ShapeDtypeStruct((_EMB_DIM, _BATCH), jnp.float32),
    )(embt, W1, b1, W2, b2t)


def kernel(book_id, table, W1, b1, W2, b2):
    embt = _sc_gather_t(book_id, table.T)
    out_t = _tc_mlp_t(embt, W1, b1.reshape(1, _HIDDEN), W2,
                      b2.reshape(_EMB_DIM, 1))
    return out_t.T


# speedup vs baseline: 1.1269x; 1.1269x over previous
import functools

import jax
import jax.numpy as jnp
from jax import lax
from jax.experimental import pallas as pl
from jax.experimental.pallas import tpu as pltpu
from jax.experimental.pallas import tpu_sc as plsc

_VOCAB = 100000
_PRE_DIM = 64
_EMB_DIM = 64
_HIDDEN = 256
_BATCH = 16384

_NC = 2
_NS = 16
_NW = _NC * _NS
_CPW = _PRE_DIM // _NW
_OCHUNK = 8192


def _sc_gather_t(idx, table_t):
    mesh = plsc.VectorSubcoreMesh(core_axis_name="c", subcore_axis_name="s")

    @functools.partial(
        pl.kernel,
        out_type=jax.ShapeDtypeStruct((_PRE_DIM, _BATCH), jnp.float32),
        mesh=mesh,
        compiler_params=pltpu.CompilerParams(needs_layout_passes=False),
        scratch_types=[
            pltpu.VMEM((_BATCH,), jnp.int32),
            pltpu.VMEM((_VOCAB + 1,), jnp.float32),
            pltpu.VMEM((_OCHUNK,), jnp.float32),
            pltpu.VMEM_SHARED((_BATCH,), jnp.int32),
            pltpu.SemaphoreType.DMA,
        ],
    )
    def k(idx_hbm, table_hbm, out_hbm, idx_v, col_v, obuf_v, idx_sp, sem):
        sid = lax.axis_index("s")
        wid = sid * _NC + lax.axis_index("c")
        col_cp = pltpu.async_copy(table_hbm.at[wid * _CPW], col_v, sem)
        @pl.when(sid == 0)
        def _():
            pltpu.sync_copy(idx_hbm, idx_sp)

        plsc.subcore_barrier()
        pltpu.sync_copy(idx_sp, idx_v)

        nhalf = _BATCH // _OCHUNK
        for c in range(_CPW):
            col = wid * _CPW + c
            col_cp.wait()
            for half in range(nhalf):

                def gather_body(i, carry, half=half):
                    for u in range(4):
                        si = pl.ds(
                            pl.multiple_of(half * _OCHUNK + i * 64 + u * 16, 16),
                            16)
                        so = pl.ds(pl.multiple_of(i * 64 + u * 16, 16), 16)
                        v = idx_v[si]
                        vi = jnp.where((v >= 1) & (v <= _VOCAB), v, 0)
                        obuf_v[so] = plsc.load_gather(col_v, [vi])
                    return carry

                lax.fori_loop(0, _OCHUNK // 64, gather_body, 0)
                pltpu.sync_copy(
                    obuf_v,
                    out_hbm.at[col].at[pl.ds(half * _OCHUNK, _OCHUNK)],
                )
            if c + 1 < _CPW:
                col_cp = pltpu.async_copy(table_hbm.at[col + 1], col_v, sem)

    return k(idx, table_t)


def _mlp_body(xt_ref, w1_ref, b1_ref, w2_ref, b2_ref, ot_ref):
    xt = xt_ref[...]
    h = jax.lax.dot_general(
        xt, w1_ref[...], (((0,), (0,)), ((), ())),
        preferred_element_type=jnp.float32)
    h = jnp.maximum(h + b1_ref[...], 0.0)
    ot = jax.lax.dot_general(
        w2_ref[...], h, (((0,), (1,)), ((), ())),
        preferred_element_type=jnp.float32)
    ot_ref[...] = ot + b2_ref[...]


def _tc_mlp_t(embt, W1, b1, W2, b2t):
    tm = 2048
    grid = (_BATCH // tm,)
    return pl.pallas_call(
        _mlp_body,
        grid=grid,
        in_specs=[
            pl.BlockSpec((_PRE_DIM, tm), lambda i: (0, i)),
            pl.BlockSpec((_PRE_DIM, _HIDDEN), lambda i: (0, 0)),
            pl.BlockSpec((1, _HIDDEN), lambda i: (0, 0)),
            pl.BlockSpec((_HIDDEN, _EMB_DIM), lambda i: (0, 0)),
            pl.BlockSpec((_EMB_DIM, 1), lambda i: (0, 0)),
        ],
        out_specs=pl.BlockSpec((_EMB_DIM, tm), lambda i: (0, i)),
        out_shape=jax.ShapeDtypeStruct((_EMB_DIM, _BATCH), jnp.float32),
    )(embt, W1, b1, W2, b2t)


def kernel(book_id, table, W1, b1, W2, b2):
    embt = _sc_gather_t(book_id, table.T)
    out_t = _tc_mlp_t(embt, W1, b1.reshape(1, _HIDDEN), W2,
                      b2.reshape(_EMB_DIM, 1))
    return out_t.T


# speedup vs baseline: 1.3340x; 1.1837x over previous
import functools

import jax
import jax.numpy as jnp
from jax import lax
from jax.experimental import pallas as pl
from jax.experimental.pallas import tpu as pltpu
from jax.experimental.pallas import tpu_sc as plsc

_VOCAB = 100000
_PRE_DIM = 64
_EMB_DIM = 64
_HIDDEN = 256
_BATCH = 16384

_NC = 2
_NS = 16
_NW = _NC * _NS
_CPW = _PRE_DIM // _NW
_OCHUNK = 8192


def _sc_gather_t(idx, table_t):
    mesh = plsc.VectorSubcoreMesh(core_axis_name="c", subcore_axis_name="s")

    @functools.partial(
        pl.kernel,
        out_type=jax.ShapeDtypeStruct((_PRE_DIM, _BATCH), jnp.float32),
        mesh=mesh,
        compiler_params=pltpu.CompilerParams(needs_layout_passes=False),
        scratch_types=[
            pltpu.VMEM((_BATCH,), jnp.int32),
            pltpu.VMEM((_VOCAB + 1,), jnp.float32),
            pltpu.VMEM((_OCHUNK,), jnp.float32),
            pltpu.VMEM_SHARED((_BATCH,), jnp.int32),
            pltpu.SemaphoreType.DMA,
        ],
    )
    def k(idx_hbm, table_hbm, out_hbm, idx_v, col_v, obuf_v, idx_sp, sem):
        sid = lax.axis_index("s")
        wid = sid * _NC + lax.axis_index("c")
        col_cp = pltpu.async_copy(table_hbm.at[wid * _CPW], col_v, sem)
        @pl.when(sid == 0)
        def _():
            pltpu.sync_copy(idx_hbm, idx_sp)

        plsc.subcore_barrier()
        pltpu.sync_copy(idx_sp, idx_v)

        nhalf = _BATCH // _OCHUNK
        for c in range(_CPW):
            col = wid * _CPW + c
            col_cp.wait()
            for half in range(nhalf):

                def gather_body(i, carry, half=half):
                    for u in range(8):
                        si = pl.ds(
                            pl.multiple_of(half * _OCHUNK + i * 128 + u * 16, 16),
                            16)
                        so = pl.ds(pl.multiple_of(i * 128 + u * 16, 16), 16)
                        v = idx_v[si]
                        vi = jnp.where((v >= 1) & (v <= _VOCAB), v, 0)
                        obuf_v[so] = plsc.load_gather(col_v, [vi])
                    return carry

                lax.fori_loop(0, _OCHUNK // 128, gather_body, 0)
                pltpu.sync_copy(
                    obuf_v,
                    out_hbm.at[col].at[pl.ds(half * _OCHUNK, _OCHUNK)],
                )
            if c + 1 < _CPW:
                col_cp = pltpu.async_copy(table_hbm.at[col + 1], col_v, sem)

    return k(idx, table_t)


def _mlp_body(xt_ref, w1_ref, b1_ref, w2_ref, b2_ref, ot_ref):
    xt = xt_ref[...]
    h = jax.lax.dot_general(
        xt, w1_ref[...], (((0,), (0,)), ((), ())),
        preferred_element_type=jnp.float32)
    h = jnp.maximum(h + b1_ref[...], 0.0)
    ot = jax.lax.dot_general(
        w2_ref[...], h, (((0,), (1,)), ((), ())),
        preferred_element_type=jnp.float32)
    ot_ref[...] = ot + b2_ref[...]


def _tc_mlp_t(embt, W1, b1, W2, b2t):
    tm = 4096
    grid = (_BATCH // tm,)
    return pl.pallas_call(
        _mlp_body,
        grid=grid,
        in_specs=[
            pl.BlockSpec((_PRE_DIM, tm), lambda i: (0, i)),
            pl.BlockSpec((_PRE_DIM, _HIDDEN), lambda i: (0, 0)),
            pl.BlockSpec((1, _HIDDEN), lambda i: (0, 0)),
            pl.BlockSpec((_HIDDEN, _EMB_DIM), lambda i: (0, 0)),
            pl.BlockSpec((_EMB_DIM, 1), lambda i: (0, 0)),
        ],
        out_specs=pl.BlockSpec((_EMB_DIM, tm), lambda i: (0, i)),
        out_shape=jax.ShapeDtypeStruct((_EMB_DIM, _BATCH), jnp.float32),
    )(embt, W1, b1, W2, b2t)


def kernel(book_id, table, W1, b1, W2, b2):
    embt = _sc_gather_t(book_id, table.T)
    out_t = _tc_mlp_t(embt, W1, b1.reshape(1, _HIDDEN), W2,
                      b2.reshape(_EMB_DIM, 1))
    return out_t.T
